# baseline (speedup 1.0000x reference)
import functools

import jax
import jax.numpy as jnp
from jax import lax
from jax.experimental import pallas as pl
from jax.experimental.pallas import tpu as pltpu
from jax.experimental.pallas import tpu_sc as plsc

D = 32
SUB = 128
K = 4
CHUNK = K * SUB
NC = 2
NS = 16
NW = NC * NS
B = 16384
S = 50
PLANE = D * B
TROW = 8 * B


def _sc_gather_tiled(table, idx):
    n_flat = idx.shape[0]
    per_w = n_flat // NW
    n_chunks = per_w // CHUNK
    n_pairs = n_chunks // 2

    mesh = plsc.VectorSubcoreMesh(core_axis_name="c", subcore_axis_name="s")

    @functools.partial(
        pl.kernel,
        mesh=mesh,
        out_type=jax.ShapeDtypeStruct((S * PLANE,), jnp.float32),
        scratch_types=[
            pltpu.VMEM((2, CHUNK), jnp.int32),
            pltpu.VMEM((CHUNK, D), jnp.float32),
            pltpu.VMEM((CHUNK, D), jnp.float32),
            pltpu.VMEM((CHUNK * D,), jnp.float32),
            pltpu.VMEM((CHUNK * D,), jnp.float32),
            pltpu.SemaphoreType.DMA,
            pltpu.SemaphoreType.DMA,
            pltpu.SemaphoreType.DMA,
            pltpu.SemaphoreType.DMA,
            pltpu.SemaphoreType.DMA,
            pltpu.SemaphoreType.DMA,
        ],
        compiler_params=pltpu.CompilerParams(
            use_tc_tiling_on_sc=False,
            needs_layout_passes=False,
            disable_bounds_checks=True,
        ),
    )
    def k(table_hbm, idx_hbm, out_hbm, idx_v, rowsa, rowsb, tilesa, tilesb,
          isem0, isem1, gsem0, gsem1, wsem0, wsem1):
        wid = lax.axis_index("s") * NC + lax.axis_index("c")
        c_base = wid * n_chunks
        rows_v = (rowsa, rowsb)
        tiles_v = (tilesa, tilesb)
        isems = (isem0, isem1)
        gsems = (gsem0, gsem1)
        wsems = (wsem0, wsem1)
        i16 = lax.iota(jnp.int32, 16)
        pats = [
            ((f0 + i16) // 8) * (K * SUB * 8) + ((f0 + i16) % 8) * SUB
            for f0 in (0, 16)
        ]

        def idx_src(c):
            return idx_hbm.at[pl.ds(c * CHUNK, CHUNK)]

        def fire_idx(c, b):
            pltpu.async_copy(idx_src(c), idx_v.at[b], isems[b])

        def wait_idx(c, b):
            pltpu.make_async_copy(idx_src(c), idx_v.at[b], isems[b]).wait()

        def fire_gathers(c, b):
            for j in range(K):
                pltpu.async_copy(
                    table_hbm.at[idx_v.at[b, pl.ds(j * SUB, SUB)]],
                    rows_v[b].at[pl.ds(j * SUB, SUB), :],
                    gsems[b],
                )

        def wait_gathers(c, b):
            for j in range(K):
                pltpu.make_async_copy(
                    table_hbm.at[idx_v.at[b, pl.ds(j * SUB, SUB)]],
                    rows_v[b].at[pl.ds(j * SUB, SUB), :],
                    gsems[b],
                ).wait()

        def out_runs(c):
            s = c // (B // CHUNK)
            tile_col = (c % (B // CHUNK)) * (CHUNK // SUB)
            base = s * PLANE + tile_col * (SUB * 8)
            return [
                (tr * (K * SUB * 8),
                 base + tr * TROW)
                for tr in range(D // 8)
            ]

        def fire_writes(c, b):
            for src_off, dst_off in out_runs(c):
                pltpu.async_copy(
                    tiles_v[b].at[pl.ds(src_off, K * SUB * 8)],
                    out_hbm.at[pl.ds(dst_off, K * SUB * 8)],
                    wsems[b],
                )

        def wait_writes(c, b):
            for src_off, dst_off in out_runs(c):
                pltpu.make_async_copy(
                    tiles_v[b].at[pl.ds(src_off, K * SUB * 8)],
                    out_hbm.at[pl.ds(dst_off, K * SUB * 8)],
                    wsems[b],
                ).wait()

        def transpose(b):
            for tc in range(K):
                def rows16(g, carry):
                    r0 = tc * SUB + g * 16
                    base = tc * (SUB * 8) + g * 16 + i16 * 0
                    for u in range(16):
                        for gi in range(2):
                            vals = rows_v[b][r0 + u, pl.ds(gi * 16, 16)]
                            plsc.store_scatter(
                                tiles_v[b], [(pats[gi] + u) + base], vals
                            )
                    return carry

                lax.fori_loop(0, SUB // 16, rows16, 0)

        fire_idx(c_base, 0)
        fire_idx(c_base + 1, 1)
        wait_idx(c_base, 0)
        fire_gathers(c_base, 0)

        def pair_body(p, carry):
            for b in range(2):
                j = p * 2 + b
                c = c_base + j
                @pl.when(j < n_chunks - 1)
                def _():
                    wait_idx(c + 1, 1 - b)
                    fire_gathers(c + 1, 1 - b)

                wait_gathers(c, b)
                @pl.when(j < n_chunks - 2)
                def _():
                    fire_idx(c + 2, b)
                @pl.when(j >= 2)
                def _():
                    wait_writes(c - 2, b)

                transpose(b)
                fire_writes(c, b)
            return carry

        lax.fori_loop(0, n_pairs, pair_body, 0)
        wait_writes(c_base + n_chunks - 2, 0)
        wait_writes(c_base + n_chunks - 1, 1)

    return k(table, idx)


def kernel(x, table):
    b, s = x.shape
    v, d = table.shape
    idx = x.T.astype(jnp.int32).reshape(-1)
    flat = _sc_gather_tiled(table, idx)
    out = flat.reshape(s, d // 8, b // SUB, 8, SUB)
    out = out.transpose(2, 4, 0, 1, 3)
    return out.reshape(b, s, d)


# speedup vs baseline: 1.0617x; 1.0617x over previous
import functools

import jax
import jax.numpy as jnp
from jax import lax
from jax.experimental import pallas as pl
from jax.experimental.pallas import tpu as pltpu
from jax.experimental.pallas import tpu_sc as plsc

D = 32
SUB = 128
K = 4
CHUNK = K * SUB
NC = 2
NS = 16
NW = NC * NS
B = 16384
S = 50
PLANE = D * B
TROW = 8 * B


def _sc_gather_tiled(table, idx):
    n_flat = idx.shape[0]
    per_w = n_flat // NW
    n_chunks = per_w // CHUNK
    n_pairs = n_chunks // 2

    mesh = plsc.VectorSubcoreMesh(core_axis_name="c", subcore_axis_name="s")

    @functools.partial(
        pl.kernel,
        mesh=mesh,
        out_type=jax.ShapeDtypeStruct((S * PLANE,), jnp.float32),
        scratch_types=[
            pltpu.VMEM((2, CHUNK), jnp.int32),
            pltpu.VMEM((CHUNK, D), jnp.float32),
            pltpu.VMEM((CHUNK, D), jnp.float32),
            pltpu.VMEM((CHUNK * D,), jnp.float32),
            pltpu.VMEM((CHUNK * D,), jnp.float32),
            pltpu.SemaphoreType.DMA,
            pltpu.SemaphoreType.DMA,
            pltpu.SemaphoreType.DMA,
            pltpu.SemaphoreType.DMA,
            pltpu.SemaphoreType.DMA,
            pltpu.SemaphoreType.DMA,
        ],
        compiler_params=pltpu.CompilerParams(
            use_tc_tiling_on_sc=False,
            needs_layout_passes=False,
            disable_bounds_checks=True,
        ),
    )
    def k(table_hbm, idx_hbm, out_hbm, idx_v, rowsa, rowsb, tilesa, tilesb,
          isem0, isem1, gsem0, gsem1, wsem0, wsem1):
        wid = lax.axis_index("s") * NC + lax.axis_index("c")
        c_base = wid * n_chunks
        rows_v = (rowsa, rowsb)
        tiles_v = (tilesa, tilesb)
        isems = (isem0, isem1)
        gsems = (gsem0, gsem1)
        wsems = (wsem0, wsem1)
        i16 = lax.iota(jnp.int32, 16)
        pats = [
            ((f0 + i16) // 8) * (K * SUB * 8) + ((f0 + i16) % 8) * SUB
            for f0 in (0, 16)
        ]

        def idx_src(c):
            return idx_hbm.at[pl.ds(c * CHUNK, CHUNK)]

        def fire_idx(c, b):
            pltpu.async_copy(idx_src(c), idx_v.at[b], isems[b])

        def wait_idx(c, b):
            pltpu.make_async_copy(idx_src(c), idx_v.at[b], isems[b]).wait()

        def fire_gathers(c, b):
            for j in range(K):
                pltpu.async_copy(
                    table_hbm.at[idx_v.at[b, pl.ds(j * SUB, SUB)]],
                    rows_v[b].at[pl.ds(j * SUB, SUB), :],
                    gsems[b],
                )

        def wait_gathers(c, b):
            for j in range(K):
                pltpu.make_async_copy(
                    table_hbm.at[idx_v.at[b, pl.ds(j * SUB, SUB)]],
                    rows_v[b].at[pl.ds(j * SUB, SUB), :],
                    gsems[b],
                ).wait()

        def out_runs(c):
            s = c // (B // CHUNK)
            tile_col = (c % (B // CHUNK)) * (CHUNK // SUB)
            base = s * PLANE + tile_col * (SUB * 8)
            return [
                (tr * (K * SUB * 8),
                 base + tr * TROW)
                for tr in range(D // 8)
            ]

        def fire_writes(c, b):
            for src_off, dst_off in out_runs(c):
                pltpu.async_copy(
                    tiles_v[b].at[pl.ds(src_off, K * SUB * 8)],
                    out_hbm.at[pl.ds(dst_off, K * SUB * 8)],
                    wsems[b],
                )

        def wait_writes(c, b):
            for src_off, dst_off in out_runs(c):
                pltpu.make_async_copy(
                    tiles_v[b].at[pl.ds(src_off, K * SUB * 8)],
                    out_hbm.at[pl.ds(dst_off, K * SUB * 8)],
                    wsems[b],
                ).wait()

        def transpose(b):
            for tc in range(K):
                def rows16(g, carry):
                    r0 = tc * SUB + g * 16
                    base = tc * (SUB * 8) + g * 16 + i16 * 0
                    vals = [
                        rows_v[b][r0 + u, pl.ds(gi * 16, 16)]
                        for u in range(16)
                        for gi in range(2)
                    ]
                    for n, v in enumerate(vals):
                        u, gi = divmod(n, 2)
                        plsc.store_scatter(
                            tiles_v[b], [(pats[gi] + u) + base], v
                        )
                    return carry

                lax.fori_loop(0, SUB // 16, rows16, 0)

        fire_idx(c_base, 0)
        fire_idx(c_base + 1, 1)
        wait_idx(c_base, 0)
        fire_gathers(c_base, 0)

        def pair_body(p, carry):
            for b in range(2):
                j = p * 2 + b
                c = c_base + j
                @pl.when(j < n_chunks - 1)
                def _():
                    wait_idx(c + 1, 1 - b)
                    fire_gathers(c + 1, 1 - b)

                wait_gathers(c, b)
                @pl.when(j < n_chunks - 2)
                def _():
                    fire_idx(c + 2, b)
                @pl.when(j >= 2)
                def _():
                    wait_writes(c - 2, b)

                transpose(b)
                fire_writes(c, b)
            return carry

        lax.fori_loop(0, n_pairs, pair_body, 0)
        wait_writes(c_base + n_chunks - 2, 0)
        wait_writes(c_base + n_chunks - 1, 1)

    return k(table, idx)


def kernel(x, table):
    b, s = x.shape
    v, d = table.shape
    idx = x.T.astype(jnp.int32).reshape(-1)
    flat = _sc_gather_tiled(table, idx)
    out = flat.reshape(s, d // 8, b // SUB, 8, SUB)
    out = out.transpose(2, 4, 0, 1, 3)
    return out.reshape(b, s, d)


# speedup vs baseline: 1.1656x; 1.0979x over previous
import functools

import jax
import jax.numpy as jnp
from jax import lax
from jax.experimental import pallas as pl
from jax.experimental.pallas import tpu as pltpu
from jax.experimental.pallas import tpu_sc as plsc

D = 32
SUB = 128
K = 4
CHUNK = K * SUB
NC = 2
NS = 16
NW = NC * NS
B = 16384
S = 50
PLANE = D * B
TROW = 8 * B


def _sc_gather_tiled(table, idx):
    n_flat = idx.shape[0]
    per_w = n_flat // NW
    n_chunks = per_w // CHUNK
    n_pairs = n_chunks // 2

    mesh = plsc.VectorSubcoreMesh(core_axis_name="c", subcore_axis_name="s")

    @functools.partial(
        pl.kernel,
        mesh=mesh,
        out_type=jax.ShapeDtypeStruct((S * PLANE,), jnp.float32),
        scratch_types=[
            pltpu.VMEM((2, CHUNK), jnp.int32),
            pltpu.VMEM((CHUNK, D), jnp.float32),
            pltpu.VMEM((CHUNK, D), jnp.float32),
            pltpu.VMEM((CHUNK * D,), jnp.float32),
            pltpu.VMEM((CHUNK * D,), jnp.float32),
            pltpu.SemaphoreType.DMA,
            pltpu.SemaphoreType.DMA,
            pltpu.SemaphoreType.DMA,
            pltpu.SemaphoreType.DMA,
            pltpu.SemaphoreType.DMA,
            pltpu.SemaphoreType.DMA,
        ],
        compiler_params=pltpu.CompilerParams(
            use_tc_tiling_on_sc=False,
            needs_layout_passes=False,
            disable_bounds_checks=True,
        ),
    )
    def k(table_hbm, idx_hbm, out_hbm, idx_v, rowsa, rowsb, tilesa, tilesb,
          isem0, isem1, gsem0, gsem1, wsem0, wsem1):
        wid = lax.axis_index("s") * NC + lax.axis_index("c")
        c_base = wid * n_chunks
        rows_v = (rowsa, rowsb)
        tiles_v = (tilesa, tilesb)
        isems = (isem0, isem1)
        gsems = (gsem0, gsem1)
        wsems = (wsem0, wsem1)
        i16 = lax.iota(jnp.int32, 16)
        pats = [
            ((f0 + i16) // 8) * (K * SUB * 8) + ((f0 + i16) % 8) * SUB
            for f0 in (0, 16)
        ]

        def idx_src(c):
            return idx_hbm.at[pl.ds(c * CHUNK, CHUNK)]

        def fire_idx(c, b):
            pltpu.async_copy(idx_src(c), idx_v.at[b], isems[b])

        def wait_idx(c, b):
            pltpu.make_async_copy(idx_src(c), idx_v.at[b], isems[b]).wait()

        def fire_gathers(c, b):
            for j in range(K):
                pltpu.async_copy(
                    table_hbm.at[idx_v.at[b, pl.ds(j * SUB, SUB)]],
                    rows_v[b].at[pl.ds(j * SUB, SUB), :],
                    gsems[b],
                )

        def wait_gathers(c, b):
            for j in range(K):
                pltpu.make_async_copy(
                    table_hbm.at[idx_v.at[b, pl.ds(j * SUB, SUB)]],
                    rows_v[b].at[pl.ds(j * SUB, SUB), :],
                    gsems[b],
                ).wait()

        def out_runs(c):
            s = c // (B // CHUNK)
            tile_col = (c % (B // CHUNK)) * (CHUNK // SUB)
            base = s * PLANE + tile_col * (SUB * 8)
            return [
                (tr * (K * SUB * 8),
                 base + tr * TROW)
                for tr in range(D // 8)
            ]

        def fire_writes(c, b):
            for src_off, dst_off in out_runs(c):
                pltpu.async_copy(
                    tiles_v[b].at[pl.ds(src_off, K * SUB * 8)],
                    out_hbm.at[pl.ds(dst_off, K * SUB * 8)],
                    wsems[b],
                )

        def wait_writes(c, b):
            for src_off, dst_off in out_runs(c):
                pltpu.make_async_copy(
                    tiles_v[b].at[pl.ds(src_off, K * SUB * 8)],
                    out_hbm.at[pl.ds(dst_off, K * SUB * 8)],
                    wsems[b],
                ).wait()

        def transpose(b):
            for tc in range(K):
                def cols16(g, carry):
                    r0 = tc * SUB + g * 16
                    rvec = r0 + i16
                    dbase = tc * (SUB * 8) + g * 16
                    vals = [
                        plsc.load_gather(rows_v[b], [rvec, i16 * 0 + f])
                        for f in range(D)
                    ]
                    for f, v in enumerate(vals):
                        tiles_v[b][
                            pl.ds(
                                (f // 8) * (K * SUB * 8)
                                + (f % 8) * SUB
                                + dbase,
                                16,
                            )
                        ] = v
                    return carry

                lax.fori_loop(0, SUB // 16, cols16, 0)

        fire_idx(c_base, 0)
        fire_idx(c_base + 1, 1)
        wait_idx(c_base, 0)
        fire_gathers(c_base, 0)

        def pair_body(p, carry):
            for b in range(2):
                j = p * 2 + b
                c = c_base + j
                @pl.when(j < n_chunks - 1)
                def _():
                    wait_idx(c + 1, 1 - b)
                    fire_gathers(c + 1, 1 - b)

                wait_gathers(c, b)
                @pl.when(j < n_chunks - 2)
                def _():
                    fire_idx(c + 2, b)
                @pl.when(j >= 2)
                def _():
                    wait_writes(c - 2, b)

                transpose(b)
                fire_writes(c, b)
            return carry

        lax.fori_loop(0, n_pairs, pair_body, 0)
        wait_writes(c_base + n_chunks - 2, 0)
        wait_writes(c_base + n_chunks - 1, 1)

    return k(table, idx)


def kernel(x, table):
    b, s = x.shape
    v, d = table.shape
    idx = x.T.astype(jnp.int32).reshape(-1)
    flat = _sc_gather_tiled(table, idx)
    out = flat.reshape(s, d // 8, b // SUB, 8, SUB)
    out = out.transpose(2, 4, 0, 1, 3)
    return out.reshape(b, s, d)


# speedup vs baseline: 1.6624x; 1.4262x over previous
import functools

import jax
import jax.numpy as jnp
from jax import lax
from jax.experimental import pallas as pl
from jax.experimental.pallas import tpu as pltpu
from jax.experimental.pallas import tpu_sc as plsc

D = 32
SUB = 128
K = 4
CHUNK = K * SUB
NC = 2
NS = 16
NW = NC * NS
B = 16384
S = 50
PLANE = D * B
TROW = 8 * B


def _sc_gather_tiled(table, idx):
    n_flat = idx.shape[0]
    per_w = n_flat // NW
    n_chunks = per_w // CHUNK
    n_pairs = n_chunks // 2

    mesh = plsc.VectorSubcoreMesh(core_axis_name="c", subcore_axis_name="s")

    @functools.partial(
        pl.kernel,
        mesh=mesh,
        out_type=jax.ShapeDtypeStruct((S * PLANE // SUB, SUB), jnp.float32),
        scratch_types=[
            pltpu.VMEM((2, CHUNK), jnp.int32),
            pltpu.VMEM((CHUNK, D), jnp.float32),
            pltpu.VMEM((CHUNK, D), jnp.float32),
            pltpu.VMEM((K * (D // 8) * 8, SUB + 1), jnp.float32),
            pltpu.VMEM((K * (D // 8) * 8, SUB + 1), jnp.float32),
            pltpu.SemaphoreType.DMA,
            pltpu.SemaphoreType.DMA,
            pltpu.SemaphoreType.DMA,
            pltpu.SemaphoreType.DMA,
            pltpu.SemaphoreType.DMA,
            pltpu.SemaphoreType.DMA,
        ],
        compiler_params=pltpu.CompilerParams(
            use_tc_tiling_on_sc=False,
            needs_layout_passes=False,
            disable_bounds_checks=True,
        ),
    )
    def k(table_hbm, idx_hbm, out_hbm, idx_v, rowsa, rowsb, tilesa, tilesb,
          isem0, isem1, gsem0, gsem1, wsem0, wsem1):
        wid = lax.axis_index("s") * NC + lax.axis_index("c")
        c_base = wid * n_chunks
        rows_v = (rowsa, rowsb)
        tiles_v = (tilesa, tilesb)
        isems = (isem0, isem1)
        gsems = (gsem0, gsem1)
        wsems = (wsem0, wsem1)
        i16 = lax.iota(jnp.int32, 16)
        pats = [
            [
                ((f0 + i16) // 8) * (K * 8) + (f0 + i16) % 8 + tc * 8
                for f0 in (0, 16)
            ]
            for tc in range(K)
        ]

        def idx_src(c):
            return idx_hbm.at[pl.ds(c * CHUNK, CHUNK)]

        def fire_idx(c, b):
            pltpu.async_copy(idx_src(c), idx_v.at[b], isems[b])

        def wait_idx(c, b):
            pltpu.make_async_copy(idx_src(c), idx_v.at[b], isems[b]).wait()

        def fire_gathers(c, b):
            for j in range(K):
                pltpu.async_copy(
                    table_hbm.at[idx_v.at[b, pl.ds(j * SUB, SUB)]],
                    rows_v[b].at[pl.ds(j * SUB, SUB), :],
                    gsems[b],
                )

        def wait_gathers(c, b):
            for j in range(K):
                pltpu.make_async_copy(
                    table_hbm.at[idx_v.at[b, pl.ds(j * SUB, SUB)]],
                    rows_v[b].at[pl.ds(j * SUB, SUB), :],
                    gsems[b],
                ).wait()

        def out_runs(c):
            s = c // (B // CHUNK)
            tile_col = (c % (B // CHUNK)) * (CHUNK // SUB)
            base = (s * PLANE + tile_col * (SUB * 8)) // SUB
            return [
                ((tr * K + tc) * 8, base + tr * (TROW // SUB) + tc * 8)
                for tr in range(D // 8)
                for tc in range(K)
            ]

        def fire_writes(c, b):
            for src_row, dst_row in out_runs(c):
                pltpu.async_copy(
                    tiles_v[b].at[pl.ds(src_row, 8), pl.ds(0, SUB)],
                    out_hbm.at[pl.ds(dst_row, 8), :],
                    wsems[b],
                )

        def wait_writes(c, b):
            for src_row, dst_row in out_runs(c):
                pltpu.make_async_copy(
                    tiles_v[b].at[pl.ds(src_row, 8), pl.ds(0, SUB)],
                    out_hbm.at[pl.ds(dst_row, 8), :],
                    wsems[b],
                ).wait()

        def transpose(b):
            for tc in range(K):
                def rows16(g, carry):
                    r0 = tc * SUB + g * 16
                    col0 = g * 16
                    vals = [
                        rows_v[b][r0 + u, pl.ds(gi * 16, 16)]
                        for u in range(16)
                        for gi in range(2)
                    ]
                    for n, v in enumerate(vals):
                        u, gi = divmod(n, 2)
                        plsc.store_scatter(
                            tiles_v[b],
                            [pats[tc][gi], i16 * 0 + (col0 + u)],
                            v,
                        )
                    return carry

                lax.fori_loop(0, SUB // 16, rows16, 0)

        fire_idx(c_base, 0)
        fire_idx(c_base + 1, 1)
        wait_idx(c_base, 0)
        fire_gathers(c_base, 0)

        def pair_body(p, carry):
            for b in range(2):
                j = p * 2 + b
                c = c_base + j
                @pl.when(j < n_chunks - 1)
                def _():
                    wait_idx(c + 1, 1 - b)
                    fire_gathers(c + 1, 1 - b)

                wait_gathers(c, b)
                @pl.when(j < n_chunks - 2)
                def _():
                    fire_idx(c + 2, b)
                @pl.when(j >= 2)
                def _():
                    wait_writes(c - 2, b)

                transpose(b)
                fire_writes(c, b)
            return carry

        lax.fori_loop(0, n_pairs, pair_body, 0)
        wait_writes(c_base + n_chunks - 2, 0)
        wait_writes(c_base + n_chunks - 1, 1)

    return k(table, idx)


def kernel(x, table):
    b, s = x.shape
    v, d = table.shape
    idx = x.T.astype(jnp.int32).reshape(-1)
    flat = _sc_gather_tiled(table, idx)
    out = flat.reshape(s, d // 8, b // SUB, 8, SUB)
    out = out.transpose(2, 4, 0, 1, 3)
    return out.reshape(b, s, d)
